# baseline (speedup 1.0000x reference)
import functools

import jax
import jax.numpy as jnp
from jax import lax
from jax.experimental import pallas as pl
from jax.experimental.pallas import tpu as pltpu
from jax.experimental.pallas import tpu_sc as plsc

_B = 16384
_D = 64
_NC = 2
_NS = 16
_NW = _NC * _NS
_ROWS_PER_W = _B // _NW
_CH = 128
_NCH = _ROWS_PER_W // _CH
_G = 16

_mesh = plsc.VectorSubcoreMesh(core_axis_name="c", subcore_axis_name="s")


@functools.partial(
    pl.kernel,
    out_type=jax.ShapeDtypeStruct((_B,), jnp.float32),
    mesh=_mesh,
    compiler_params=pltpu.CompilerParams(needs_layout_passes=False),
    scratch_types=[
        pltpu.VMEM((_CH * _D,), jnp.float32),
        pltpu.VMEM((_CH * _D,), jnp.float32),
        pltpu.VMEM((_CH * _D,), jnp.float32),
        pltpu.VMEM((_CH * _D,), jnp.float32),
        pltpu.VMEM((_CH * _D,), jnp.float32),
        pltpu.VMEM((_CH * _D,), jnp.float32),
        pltpu.VMEM((_CH * _D,), jnp.float32),
        pltpu.VMEM((_CH * _D,), jnp.float32),
        pltpu.VMEM((_CH,), jnp.float32),
        pltpu.VMEM((_CH,), jnp.float32),
        pltpu.SemaphoreType.DMA,
        pltpu.SemaphoreType.DMA,
    ],
)
def _sc_dot_kernel(
    gu, gi, gut, git, out,
    bu0, bi0, but0, bit0, bu1, bi1, but1, bit1, bout0, bout1,
    sem_in, sem_out,
):
    wid = lax.axis_index("s") * _NC + lax.axis_index("c")
    base = wid * _ROWS_PER_W
    lane = lax.iota(jnp.int32, _G)
    diag = lane * _D + lane

    bufs = ((bu0, bi0, but0, bit0), (bu1, bi1, but1, bit1))
    bouts = (bout0, bout1)
    sz = _CH * _D

    def fire(c, slot):
        r0 = (base + c * _CH) * _D
        cu, ci, cut, cit = bufs[slot]
        pltpu.async_copy(gu.at[pl.ds(r0, sz)], cu, sem_in)
        pltpu.async_copy(gi.at[pl.ds(r0, sz)], ci, sem_in)
        pltpu.async_copy(gut.at[pl.ds(r0, sz)], cut, sem_in)
        pltpu.async_copy(git.at[pl.ds(r0, sz)], cit, sem_in)

    def drain(slot):
        cu, ci, cut, cit = bufs[slot]
        pltpu.make_async_copy(gu.at[pl.ds(0, sz)], cu, sem_in).wait()
        pltpu.make_async_copy(gi.at[pl.ds(0, sz)], ci, sem_in).wait()
        pltpu.make_async_copy(gut.at[pl.ds(0, sz)], cut, sem_in).wait()
        pltpu.make_async_copy(git.at[pl.ds(0, sz)], cit, sem_in).wait()

    fire(0, 0)

    for c in range(_NCH):
        slot = c % 2
        drain(slot)
        if c + 1 < _NCH:
            fire(c + 1, 1 - slot)

        cu, ci, cut, cit = bufs[slot]
        bout = bouts[slot]

        def group_body(g, _):
            gbase = diag + g * (_G * _D)
            a0 = [None] * 4
            a1 = [None] * 4
            for d in range(_D):
                if d == 0:
                    idx = gbase
                else:
                    idx = gbase + jnp.where(lane >= _D - d, d - _D, d)
                j = d % 4
                p0 = plsc.load_gather(cu, [idx]) * plsc.load_gather(ci, [idx])
                p1 = plsc.load_gather(cut, [idx]) * plsc.load_gather(cit, [idx])
                a0[j] = p0 if a0[j] is None else a0[j] + p0
                a1[j] = p1 if a1[j] is None else a1[j] + p1
            bout[pl.ds(g * _G, _G)] = (
                ((a0[0] + a0[1]) + (a0[2] + a0[3]))
                + ((a1[0] + a1[1]) + (a1[2] + a1[3]))
            )
            return ()

        lax.fori_loop(0, _CH // _G, group_body, ())

        r0 = base + c * _CH
        if c >= 2:
            pltpu.make_async_copy(bouts[1 - slot], out.at[pl.ds(0, _CH)], sem_out).wait()
        pltpu.async_copy(bout, out.at[pl.ds(r0, _CH)], sem_out)

    pltpu.make_async_copy(bouts[0], out.at[pl.ds(0, _CH)], sem_out).wait()
    pltpu.make_async_copy(bouts[1], out.at[pl.ds(0, _CH)], sem_out).wait()


def kernel(gu, gi, gut, git):
    gu_f = gu.reshape(_B * _D)
    gi_f = gi.reshape(_B * _D)
    gut_f = gut.reshape(_B * _D)
    git_f = git.reshape(_B * _D)
    return _sc_dot_kernel(gu_f, gi_f, gut_f, git_f)


# speedup vs baseline: 1.7301x; 1.7301x over previous
import functools

import jax
import jax.numpy as jnp
from jax import lax
from jax.experimental import pallas as pl
from jax.experimental.pallas import tpu as pltpu
from jax.experimental.pallas import tpu_sc as plsc

_B = 16384
_D = 64
_NC = 2
_NS = 16
_NW = _NC * _NS
_ROWS_PER_W = _B // _NW
_CH = 64
_NCH = _ROWS_PER_W // _CH
_G = 16

_mesh = plsc.VectorSubcoreMesh(core_axis_name="c", subcore_axis_name="s")


@functools.partial(
    pl.kernel,
    out_type=jax.ShapeDtypeStruct((_B,), jnp.float32),
    mesh=_mesh,
    compiler_params=pltpu.CompilerParams(needs_layout_passes=False),
    scratch_types=[
        pltpu.VMEM((_CH, _D), jnp.float32),
        pltpu.VMEM((_CH, _D), jnp.float32),
        pltpu.VMEM((_CH, _D), jnp.float32),
        pltpu.VMEM((_CH, _D), jnp.float32),
        pltpu.VMEM((_CH, _D), jnp.float32),
        pltpu.VMEM((_CH, _D), jnp.float32),
        pltpu.VMEM((_CH, _D), jnp.float32),
        pltpu.VMEM((_CH, _D), jnp.float32),
        pltpu.VMEM((_CH,), jnp.float32),
        pltpu.VMEM((_CH,), jnp.float32),
        pltpu.SemaphoreType.DMA,
        pltpu.SemaphoreType.DMA,
    ],
)
def _sc_dot_kernel(
    gu, gi, gut, git, out,
    bu0, bi0, but0, bit0, bu1, bi1, but1, bit1, bout0, bout1,
    sem_in, sem_out,
):
    wid = lax.axis_index("s") * _NC + lax.axis_index("c")
    base = wid * _ROWS_PER_W
    lane = lax.iota(jnp.int32, _G)

    bufs = ((bu0, bi0, but0, bit0), (bu1, bi1, but1, bit1))
    bouts = (bout0, bout1)

    def fire(c, slot):
        r0 = base + c * _CH
        cu, ci, cut, cit = bufs[slot]
        pltpu.async_copy(gu.at[pl.ds(r0, _CH), :], cu, sem_in)
        pltpu.async_copy(gi.at[pl.ds(r0, _CH), :], ci, sem_in)
        pltpu.async_copy(gut.at[pl.ds(r0, _CH), :], cut, sem_in)
        pltpu.async_copy(git.at[pl.ds(r0, _CH), :], cit, sem_in)

    def drain(slot):
        cu, ci, cut, cit = bufs[slot]
        pltpu.make_async_copy(gu.at[pl.ds(0, _CH), :], cu, sem_in).wait()
        pltpu.make_async_copy(gi.at[pl.ds(0, _CH), :], ci, sem_in).wait()
        pltpu.make_async_copy(gut.at[pl.ds(0, _CH), :], cut, sem_in).wait()
        pltpu.make_async_copy(git.at[pl.ds(0, _CH), :], cit, sem_in).wait()

    fire(0, 0)

    for c in range(_NCH):
        slot = c % 2
        drain(slot)
        if c + 1 < _NCH:
            fire(c + 1, 1 - slot)

        cu, ci, cut, cit = bufs[slot]
        bout = bouts[slot]

        def group_body(g, _):
            row = g * _G + lane
            zero = jnp.zeros((_G,), jnp.float32)

            def d_body(d, carry):
                acc0, acc1, col = carry
                acc0 += plsc.load_gather(cu, [row, col]) * plsc.load_gather(
                    ci, [row, col]
                )
                acc1 += plsc.load_gather(cut, [row, col]) * plsc.load_gather(
                    cit, [row, col]
                )
                col += 1
                col = jnp.where(col == _D, 0, col)
                return acc0, acc1, col

            acc0, acc1, _ = lax.fori_loop(
                0, _D, d_body, (zero, zero, lane), unroll=8
            )
            bout[pl.ds(g * _G, _G)] = acc0 + acc1
            return ()

        lax.fori_loop(0, _CH // _G, group_body, ())

        r0 = base + c * _CH
        if c >= 2:
            pltpu.make_async_copy(bouts[1 - slot], out.at[pl.ds(0, _CH)], sem_out).wait()
        pltpu.async_copy(bout, out.at[pl.ds(r0, _CH)], sem_out)

    pltpu.make_async_copy(bouts[0], out.at[pl.ds(0, _CH)], sem_out).wait()
    pltpu.make_async_copy(bouts[1], out.at[pl.ds(0, _CH)], sem_out).wait()


def kernel(gu, gi, gut, git):
    return _sc_dot_kernel(gu, gi, gut, git)


# speedup vs baseline: 1.7376x; 1.0043x over previous
import functools

import jax
import jax.numpy as jnp
from jax import lax
from jax.experimental import pallas as pl
from jax.experimental.pallas import tpu as pltpu
from jax.experimental.pallas import tpu_sc as plsc

_B = 16384
_D = 64
_NC = 2
_NS = 16
_NW = _NC * _NS
_ROWS_PER_W = _B // _NW
_CH = 64
_NCH = _ROWS_PER_W // _CH
_G = 16

_mesh = plsc.VectorSubcoreMesh(core_axis_name="c", subcore_axis_name="s")


@functools.partial(
    pl.kernel,
    out_type=jax.ShapeDtypeStruct((_B,), jnp.float32),
    mesh=_mesh,
    compiler_params=pltpu.CompilerParams(needs_layout_passes=False, use_tc_tiling_on_sc=True),
    scratch_types=[
        pltpu.VMEM((_CH, _D), jnp.float32),
        pltpu.VMEM((_CH, _D), jnp.float32),
        pltpu.VMEM((_CH, _D), jnp.float32),
        pltpu.VMEM((_CH, _D), jnp.float32),
        pltpu.VMEM((_CH, _D), jnp.float32),
        pltpu.VMEM((_CH, _D), jnp.float32),
        pltpu.VMEM((_CH, _D), jnp.float32),
        pltpu.VMEM((_CH, _D), jnp.float32),
        pltpu.VMEM((_CH,), jnp.float32),
        pltpu.VMEM((_CH,), jnp.float32),
        pltpu.SemaphoreType.DMA,
        pltpu.SemaphoreType.DMA,
    ],
)
def _sc_dot_kernel(
    gu, gi, gut, git, out,
    bu0, bi0, but0, bit0, bu1, bi1, but1, bit1, bout0, bout1,
    sem_in, sem_out,
):
    wid = lax.axis_index("s") * _NC + lax.axis_index("c")
    base = wid * _ROWS_PER_W
    lane = lax.iota(jnp.int32, _G)

    bufs = ((bu0, bi0, but0, bit0), (bu1, bi1, but1, bit1))
    bouts = (bout0, bout1)

    def fire(c, slot):
        r0 = base + c * _CH
        cu, ci, cut, cit = bufs[slot]
        pltpu.async_copy(gu.at[pl.ds(r0, _CH), :], cu, sem_in)
        pltpu.async_copy(gi.at[pl.ds(r0, _CH), :], ci, sem_in)
        pltpu.async_copy(gut.at[pl.ds(r0, _CH), :], cut, sem_in)
        pltpu.async_copy(git.at[pl.ds(r0, _CH), :], cit, sem_in)

    def drain(slot):
        cu, ci, cut, cit = bufs[slot]
        pltpu.make_async_copy(gu.at[pl.ds(0, _CH), :], cu, sem_in).wait()
        pltpu.make_async_copy(gi.at[pl.ds(0, _CH), :], ci, sem_in).wait()
        pltpu.make_async_copy(gut.at[pl.ds(0, _CH), :], cut, sem_in).wait()
        pltpu.make_async_copy(git.at[pl.ds(0, _CH), :], cit, sem_in).wait()

    fire(0, 0)

    for c in range(_NCH):
        slot = c % 2
        drain(slot)
        if c + 1 < _NCH:
            fire(c + 1, 1 - slot)

        cu, ci, cut, cit = bufs[slot]
        bout = bouts[slot]

        def group_body(g, _):
            row = g * _G + lane
            zero = jnp.zeros((_G,), jnp.float32)

            def d_body(d, carry):
                acc0, acc1, col = carry
                acc0 += plsc.load_gather(cu, [row, col]) * plsc.load_gather(
                    ci, [row, col]
                )
                acc1 += plsc.load_gather(cut, [row, col]) * plsc.load_gather(
                    cit, [row, col]
                )
                col += 1
                col = jnp.where(col == _D, 0, col)
                return acc0, acc1, col

            acc0, acc1, _ = lax.fori_loop(
                0, _D, d_body, (zero, zero, lane), unroll=8
            )
            bout[pl.ds(g * _G, _G)] = acc0 + acc1
            return ()

        lax.fori_loop(0, _CH // _G, group_body, ())

        r0 = base + c * _CH
        if c >= 2:
            pltpu.make_async_copy(bouts[1 - slot], out.at[pl.ds(0, _CH)], sem_out).wait()
        pltpu.async_copy(bout, out.at[pl.ds(r0, _CH)], sem_out)

    pltpu.make_async_copy(bouts[0], out.at[pl.ds(0, _CH)], sem_out).wait()
    pltpu.make_async_copy(bouts[1], out.at[pl.ds(0, _CH)], sem_out).wait()


def kernel(gu, gi, gut, git):
    return _sc_dot_kernel(gu, gi, gut, git)
